# baseline (speedup 1.0000x reference)
import functools

import jax
import jax.numpy as jnp
from jax import lax
from jax.experimental import pallas as pl
from jax.experimental.pallas import tpu as pltpu
from jax.experimental.pallas import tpu_sc as plsc

_N_NEURONS = 53
_N_SF = 128
_LANES = 16


def _sc_gather_softmax(neuron_list, alphas):
    (b,) = neuron_list.shape
    info = plsc.get_sparse_core_info()
    nc, ns = info.num_cores, info.num_subcores
    nw = nc * ns
    assert b % (8 * nw) == 0
    b_per_w = b // nw
    n_chunks = _N_SF // _LANES

    mesh = plsc.VectorSubcoreMesh(core_axis_name="c", subcore_axis_name="s")

    @functools.partial(
        pl.kernel,
        mesh=mesh,
        out_type=jax.ShapeDtypeStruct((b, _N_SF), jnp.float32),
        scratch_types=[
            pltpu.VMEM((b_per_w,), jnp.int32),
            pltpu.VMEM((b_per_w, _N_SF), jnp.float32),
            pltpu.SemaphoreType.DMA,
        ],
        compiler_params=pltpu.CompilerParams(needs_layout_passes=False),
    )
    def k(idx_hbm, alphas_hbm, att_hbm, idx_v, rows_v, sem):
        wid = lax.axis_index("s") * nc + lax.axis_index("c")
        base = wid * b_per_w
        pltpu.sync_copy(idx_hbm.at[pl.ds(base, b_per_w)], idx_v)
        pltpu.async_copy(alphas_hbm.at[idx_v], rows_v, sem).wait()

        def row_softmax(r, carry):
            chunks = [
                jnp.exp(rows_v[r, pl.ds(c * _LANES, _LANES)])
                for c in range(n_chunks)
            ]
            acc = chunks[0]
            for c in range(1, n_chunks):
                acc = acc + chunks[c]
            inv = 1.0 / jnp.full((_LANES,), jnp.sum(acc), jnp.float32)
            for c in range(n_chunks):
                rows_v[r, pl.ds(c * _LANES, _LANES)] = chunks[c] * inv
            return carry

        lax.fori_loop(0, b_per_w, row_softmax, 0)

        pltpu.sync_copy(rows_v, att_hbm.at[pl.ds(base, b_per_w)])

    return k(neuron_list, alphas)


_BB = 64
_B0 = 256


def _wsum(x_blk, att_blk, cs):
    t = x_blk * att_blk[:, None, None, :]
    return jnp.sum(t, axis=3).reshape(x_blk.shape[0], cs)


def _tc_first(neuron_list, alphas, xt, cs):
    n = alphas.shape[1]
    nl3 = neuron_list[:_B0].reshape(_B0 // _BB, 1, _BB)
    alphas_pad = jnp.zeros((n, n), jnp.float32).at[: alphas.shape[0]].set(alphas)

    def body(nl_ref, a_ref, x_ref, o_ref):
        ids = nl_ref[0, :, :]
        row_iota = lax.broadcasted_iota(jnp.int32, (n, _BB), 0)
        onehot_t = (row_iota == jnp.broadcast_to(ids, (n, _BB))).astype(
            jnp.float32
        )
        g = lax.dot_general(
            onehot_t, a_ref[...], (((0,), (0,)), ((), ())),
            preferred_element_type=jnp.float32,
        )
        e = jnp.exp(g)
        att_blk = e / jnp.sum(e, axis=1, keepdims=True)
        o_ref[...] = _wsum(x_ref[...], att_blk, cs)

    return pl.pallas_call(
        body,
        grid=(_B0 // _BB,),
        in_specs=[
            pl.BlockSpec((1, 1, _BB), lambda i: (i, 0, 0)),
            pl.BlockSpec((n, n), lambda i: (0, 0)),
            pl.BlockSpec((_BB, 32, cs // 32, n), lambda i: (i, 0, 0, 0)),
        ],
        out_specs=pl.BlockSpec((_BB, cs), lambda i: (i, 0)),
        out_shape=jax.ShapeDtypeStruct((_B0, cs), jnp.float32),
    )(nl3, alphas_pad, xt)


def _tc_main(att, xt, cs):
    b, n = att.shape
    nblk = (b - _B0) // _BB
    off = _B0 // _BB

    def body(att_ref, x_ref, o_ref):
        o_ref[...] = _wsum(x_ref[...], att_ref[...], cs)

    return pl.pallas_call(
        body,
        grid=(nblk,),
        in_specs=[
            pl.BlockSpec((_BB, n), lambda i: (i + off, 0)),
            pl.BlockSpec((_BB, 32, cs // 32, n), lambda i: (i + off, 0, 0, 0)),
        ],
        out_specs=pl.BlockSpec((_BB, cs), lambda i: (i, 0)),
        out_shape=jax.ShapeDtypeStruct((b - _B0, cs), jnp.float32),
    )(att, xt)


def kernel(x, neuron_list, alphas):
    b, n, c, s = x.shape
    cs = c * s
    xt = jnp.transpose(x, (0, 2, 3, 1))
    att = _sc_gather_softmax(neuron_list, alphas)
    out_first = _tc_first(neuron_list, alphas, xt, cs)
    out_main = _tc_main(att, xt, cs)
    out = jnp.concatenate([out_first, out_main], axis=0)
    return out, att


# speedup vs baseline: 1.0919x; 1.0919x over previous
import functools

import jax
import jax.numpy as jnp
from jax import lax
from jax.experimental import pallas as pl
from jax.experimental.pallas import tpu as pltpu
from jax.experimental.pallas import tpu_sc as plsc

_BB = 64


def _sc_gather(neuron_list, alphas):
    (b,) = neuron_list.shape
    n = alphas.shape[1]
    info = plsc.get_sparse_core_info()
    nc, ns = info.num_cores, info.num_subcores
    nw = nc * ns
    assert b % (8 * nw) == 0
    b_per_w = b // nw

    mesh = plsc.VectorSubcoreMesh(core_axis_name="c", subcore_axis_name="s")

    @functools.partial(
        pl.kernel,
        mesh=mesh,
        out_type=jax.ShapeDtypeStruct((b, n), jnp.float32),
        scratch_types=[
            pltpu.VMEM((b_per_w,), jnp.int32),
            pltpu.VMEM((b_per_w, n), jnp.float32),
            pltpu.SemaphoreType.DMA,
        ],
        compiler_params=pltpu.CompilerParams(needs_layout_passes=False),
    )
    def k(idx_hbm, alphas_hbm, g_hbm, idx_v, rows_v, sem):
        wid = lax.axis_index("s") * nc + lax.axis_index("c")
        base = wid * b_per_w
        pltpu.sync_copy(idx_hbm.at[pl.ds(base, b_per_w)], idx_v)
        pltpu.async_copy(alphas_hbm.at[idx_v], rows_v, sem).wait()
        pltpu.sync_copy(rows_v, g_hbm.at[pl.ds(base, b_per_w)])

    return k(neuron_list, alphas)


def _tc_softmax_wsum(g, xt, cs):
    b, n = g.shape

    def body(g_ref, x_ref, o_ref, att_ref):
        e = jnp.exp(g_ref[...])
        att_blk = e / jnp.sum(e, axis=1, keepdims=True)
        att_ref[...] = att_blk
        t = x_ref[...] * att_blk[:, None, None, :]
        o_ref[...] = jnp.sum(t, axis=3).reshape(_BB, cs)

    return pl.pallas_call(
        body,
        grid=(b // _BB,),
        in_specs=[
            pl.BlockSpec((_BB, n), lambda i: (i, 0)),
            pl.BlockSpec((_BB, 32, cs // 32, n), lambda i: (i, 0, 0, 0)),
        ],
        out_specs=[
            pl.BlockSpec((_BB, cs), lambda i: (i, 0)),
            pl.BlockSpec((_BB, n), lambda i: (i, 0)),
        ],
        out_shape=[
            jax.ShapeDtypeStruct((b, cs), jnp.float32),
            jax.ShapeDtypeStruct((b, n), jnp.float32),
        ],
    )(g, xt)


def kernel(x, neuron_list, alphas):
    b, n, c, s = x.shape
    cs = c * s
    xt = jnp.transpose(x, (0, 2, 3, 1))
    g = _sc_gather(neuron_list, alphas)
    out, att = _tc_softmax_wsum(g, xt, cs)
    return out, att
